# baseline (speedup 1.0000x reference)
import functools

import jax
import jax.numpy as jnp
from jax import lax
from jax.experimental import pallas as pl
from jax.experimental.pallas import tpu as pltpu
from jax.experimental.pallas import tpu_sc as plsc

K = 8
B, D, N = 4, 2048, 8192
R = B * D
L = 16
VPR = N // L
NC, NS = 2, 16
NW = NC * NS
RPW = R // NW
CR = 4
CW = CR * N
NCH = RPW // CR
NSLOT = N + 32
NEG = float("-inf")
DELTA = 0.25


def _sortd(v):
    sk, _ = plsc.sort_key_val(v, v, descending=True)
    return sk


def _msort(a, b, lane):
    comb = jnp.where(lane < K, a, lax.rev(b, (0,)))
    return _sortd(comb)


def _scalar0(v):
    return lax.squeeze(lax.slice(v, (0,), (1,)), (0,))


def _sc_body(x_hbm, out_hbm, buf, slots, stage, nslot, test_ref, sem0, sem1):
    cid = lax.axis_index("c")
    sid = lax.axis_index("s")
    w = sid * NC + cid
    row0 = w * RPW
    base_off = row0 * N
    b_idx = w // (D // RPW)
    d0 = (w % (D // RPW)) * RPW

    lane = lax.iota(jnp.int32, L)
    lt8 = lane < K
    neg_v = jnp.full((L,), NEG, jnp.float32)

    def copy(c, par, sem):
        return pltpu.make_async_copy(
            x_hbm.at[pl.ds(base_off + c * CW, CW)],
            buf.at[pl.ds(par * CW, CW)],
            sem,
        )

    copy(0, 0, sem0).start()
    copy(1, 1, sem1).start()
    test_ref[0] = jnp.float32(jnp.inf)

    def row_scan(rb, t_s):
        t_vec = jnp.full((L,), t_s, jnp.float32)

        def body(i, carry):
            m_run, ptr_v = carry
            x = buf[pl.ds(rb + i * L, L)]
            m_run = jnp.maximum(m_run, x)
            mask = x >= t_vec
            pre = plsc.cumsum(mask.astype(jnp.int32))
            idx = ptr_v + pre - 1
            plsc.store_scatter(slots, [idx], x, mask=mask)
            cnt = plsc.all_reduce_population_count(mask)
            return m_run, ptr_v + cnt

        zero_v = jnp.zeros((L,), jnp.int32)
        m_run, ptr_v = lax.fori_loop(
            0, VPR, body, (neg_v, zero_v), unroll=8)
        return m_run, _scalar0(ptr_v)

    def chunk_body(c, carry):
        par = c & 1
        pbase = par * CW

        @pl.when(par == 0)
        def _():
            copy(c, 0, sem0).wait()

        @pl.when(par == 1)
        def _():
            copy(c, 1, sem1).wait()

        def row_body(r, _):
            rb = pbase + r * N
            t_est = test_ref[0]
            m_run, ptr = row_scan(rb, t_est)

            sm = _sortd(m_run)
            t_ex = jnp.max(jnp.where(lane == K - 1, sm, NEG))
            test_ref[0] = t_ex - jnp.float32(DELTA)
            nslot[0] = ptr

            @pl.when(t_est > t_ex)
            def _():
                _, p2 = row_scan(rb, t_ex)
                nslot[0] = p2

            ptr_f = nslot[0]
            slots[pl.ds(ptr_f, L)] = neg_v
            nq = (ptr_f + 15) >> 4

            def tmerge(ci, acc):
                s = _sortd(slots[pl.ds(ci * L, L)])
                return _msort(acc, s, lane)

            top8 = lax.fori_loop(0, nq, tmerge, neg_v)

            i_row = c * CR + r
            plsc.store_scatter(stage, [lane * RPW + i_row], top8, mask=lt8)
            return 0

        lax.fori_loop(0, CR, row_body, 0)

        c2 = c + 2

        @pl.when((c2 < NCH) & (par == 0))
        def _():
            copy(c2, 0, sem0).start()

        @pl.when((c2 < NCH) & (par == 1))
        def _():
            copy(c2, 1, sem1).start()

        return carry

    lax.fori_loop(0, NCH, chunk_body, 0)

    for j in range(K):
        pltpu.sync_copy(
            stage.at[pl.ds(j * RPW, RPW)],
            out_hbm.at[b_idx, j, pl.ds(d0, RPW)],
        )


@functools.partial(
    pl.kernel,
    out_type=jax.ShapeDtypeStruct((B, K, D), jnp.float32),
    mesh=plsc.VectorSubcoreMesh(core_axis_name="c", subcore_axis_name="s"),
    compiler_params=pltpu.CompilerParams(needs_layout_passes=False),
    scratch_types=[
        pltpu.VMEM((2 * CW,), jnp.float32),
        pltpu.VMEM((NSLOT,), jnp.float32),
        pltpu.VMEM((K * RPW,), jnp.float32),
        pltpu.SMEM((1,), jnp.int32),
        pltpu.SMEM((1,), jnp.float32),
        pltpu.SemaphoreType.DMA,
        pltpu.SemaphoreType.DMA,
    ],
)
def _sc_topk(x_hbm, out_hbm, buf, slots, stage, nslot, test_ref, sem0, sem1):
    _sc_body(x_hbm, out_hbm, buf, slots, stage, nslot, test_ref, sem0, sem1)


def kernel(inputs):
    return _sc_topk(inputs.reshape(-1))


# speedup vs baseline: 4.3450x; 4.3450x over previous
import functools

import jax
import jax.numpy as jnp
from jax import lax
from jax.experimental import pallas as pl
from jax.experimental.pallas import tpu as pltpu
from jax.experimental.pallas import tpu_sc as plsc

K = 8
B, D, N = 4, 2048, 8192
R = B * D
L = 16
VPR = N // L
G = 8
NG = VPR // G
NC, NS = 2, 16
NW = NC * NS
RPW = R // NW
CR = 4
CW = CR * N
NCH = RPW // CR
NEG = float("-inf")

_SORT8 = [(0, 1), (2, 3), (4, 5), (6, 7),
          (0, 2), (1, 3), (4, 6), (5, 7),
          (1, 2), (5, 6), (0, 4), (3, 7),
          (1, 5), (2, 6),
          (1, 4), (3, 6),
          (2, 4), (3, 5),
          (3, 4)]
_CLEAN8 = [(0, 4), (1, 5), (2, 6), (3, 7),
           (0, 2), (1, 3), (4, 6), (5, 7),
           (0, 1), (2, 3), (4, 5), (6, 7)]


def _sortd(v):
    sk, _ = plsc.sort_key_val(v, v, descending=True)
    return sk


def _msort(a, b, lane):
    comb = jnp.where(lane < K, a, lax.rev(b, (0,)))
    return _sortd(comb)


def _net(vs, pairs):
    vs = list(vs)
    for i, j in pairs:
        hi = jnp.maximum(vs[i], vs[j])
        lo = jnp.minimum(vs[i], vs[j])
        vs[i], vs[j] = hi, lo
    return vs


def _merge_topk(s, xs):
    r = [jnp.maximum(s[i], xs[K - 1 - i]) for i in range(K)]
    return _net(r, _CLEAN8)


def _sc_body(x_hbm, out_hbm, buf, stage, sem0, sem1):
    cid = lax.axis_index("c")
    sid = lax.axis_index("s")
    w = sid * NC + cid
    row0 = w * RPW
    base_off = row0 * N
    b_idx = w // (D // RPW)
    d0 = (w % (D // RPW)) * RPW

    lane = lax.iota(jnp.int32, L)
    lt8 = lane < K
    neg_v = jnp.full((L,), NEG, jnp.float32)

    def copy(c, par, sem):
        return pltpu.make_async_copy(
            x_hbm.at[pl.ds(base_off + c * CW, CW)],
            buf.at[pl.ds(par * CW, CW)],
            sem,
        )

    copy(0, 0, sem0).start()
    copy(1, 1, sem1).start()

    def chunk_body(c, carry):
        par = c & 1
        pbase = par * CW

        @pl.when(par == 0)
        def _():
            copy(c, 0, sem0).wait()

        @pl.when(par == 1)
        def _():
            copy(c, 1, sem1).wait()

        def row_body(r, _):
            rb = pbase + r * N

            def pair_body(p, carrys):
                sa, sb = carrys
                ga = rb + p * (2 * G * L)
                xa = [buf[pl.ds(ga + i * L, L)] for i in range(G)]
                sa = _merge_topk(sa, _net(xa, _SORT8))
                gb = ga + G * L
                xb = [buf[pl.ds(gb + i * L, L)] for i in range(G)]
                sb = _merge_topk(sb, _net(xb, _SORT8))
                return tuple(sa), tuple(sb)

            init = (neg_v,) * K
            sa, sb = lax.fori_loop(0, NG // 2, pair_body, (init, init))

            s = _merge_topk(list(sa), list(sb))
            t01 = _msort(_sortd(s[0]), _sortd(s[1]), lane)
            t23 = _msort(_sortd(s[2]), _sortd(s[3]), lane)
            t45 = _msort(_sortd(s[4]), _sortd(s[5]), lane)
            t67 = _msort(_sortd(s[6]), _sortd(s[7]), lane)
            top8 = _msort(_msort(t01, t23, lane), _msort(t45, t67, lane),
                          lane)

            i_row = c * CR + r
            plsc.store_scatter(stage, [lane * RPW + i_row], top8, mask=lt8)
            return 0

        lax.fori_loop(0, CR, row_body, 0)

        c2 = c + 2

        @pl.when((c2 < NCH) & (par == 0))
        def _():
            copy(c2, 0, sem0).start()

        @pl.when((c2 < NCH) & (par == 1))
        def _():
            copy(c2, 1, sem1).start()

        return carry

    lax.fori_loop(0, NCH, chunk_body, 0)

    for j in range(K):
        pltpu.sync_copy(
            stage.at[pl.ds(j * RPW, RPW)],
            out_hbm.at[b_idx, j, pl.ds(d0, RPW)],
        )


@functools.partial(
    pl.kernel,
    out_type=jax.ShapeDtypeStruct((B, K, D), jnp.float32),
    mesh=plsc.VectorSubcoreMesh(core_axis_name="c", subcore_axis_name="s"),
    compiler_params=pltpu.CompilerParams(needs_layout_passes=False),
    scratch_types=[
        pltpu.VMEM((2 * CW,), jnp.float32),
        pltpu.VMEM((K * RPW,), jnp.float32),
        pltpu.SemaphoreType.DMA,
        pltpu.SemaphoreType.DMA,
    ],
)
def _sc_topk(x_hbm, out_hbm, buf, stage, sem0, sem1):
    _sc_body(x_hbm, out_hbm, buf, stage, sem0, sem1)


def kernel(inputs):
    return _sc_topk(inputs.reshape(-1))


# speedup vs baseline: 7.3763x; 1.6977x over previous
import functools

import jax
import jax.numpy as jnp
from jax import lax
from jax.experimental import pallas as pl
from jax.experimental.pallas import tpu as pltpu
from jax.experimental.pallas import tpu_sc as plsc

K = 8
B, D, N = 4, 2048, 8192
R = B * D
L = 16
VPR = N // L
G = 8
NG = VPR // G
NC, NS = 2, 16
NW = NC * NS
RPW = R // NW
CR = 4
CW = CR * N
NCH = RPW // CR
NEG = float("-inf")

_SORT8 = [(0, 1), (2, 3), (4, 5), (6, 7),
          (0, 2), (1, 3), (4, 6), (5, 7),
          (1, 2), (5, 6), (0, 4), (3, 7),
          (1, 5), (2, 6),
          (1, 4), (3, 6),
          (2, 4), (3, 5),
          (3, 4)]
_CLEAN8 = [(0, 4), (1, 5), (2, 6), (3, 7),
           (0, 2), (1, 3), (4, 6), (5, 7),
           (0, 1), (2, 3), (4, 5), (6, 7)]


def _sortd(v):
    sk, _ = plsc.sort_key_val(v, v, descending=True)
    return sk


def _msort(a, b, lane):
    comb = jnp.where(lane < K, a, lax.rev(b, (0,)))
    return _sortd(comb)


def _net(vs, pairs):
    vs = list(vs)
    for i, j in pairs:
        hi = jnp.maximum(vs[i], vs[j])
        lo = jnp.minimum(vs[i], vs[j])
        vs[i], vs[j] = hi, lo
    return vs


def _merge_topk(s, xs):
    r = [jnp.maximum(s[i], xs[K - 1 - i]) for i in range(K)]
    return _net(r, _CLEAN8)


def _sc_body(x_hbm, out_hbm, buf, stage, sem0, sem1):
    cid = lax.axis_index("c")
    sid = lax.axis_index("s")
    w = sid * NC + cid
    row0 = w * RPW
    base_off = row0 * N
    b_idx = w // (D // RPW)
    d0 = (w % (D // RPW)) * RPW

    lane = lax.iota(jnp.int32, L)
    lt8 = lane < K
    neg_v = jnp.full((L,), NEG, jnp.float32)

    def copy(c, par, sem):
        return pltpu.make_async_copy(
            x_hbm.at[b_idx, pl.ds(d0 + c * CR, CR), :],
            buf.at[par],
            sem,
        )

    copy(0, 0, sem0).start()
    copy(1, 1, sem1).start()

    def chunk_body(c, carry):
        par = c & 1

        @pl.when(par == 0)
        def _():
            copy(c, 0, sem0).wait()

        @pl.when(par == 1)
        def _():
            copy(c, 1, sem1).wait()

        def row_body(r, _):
            def pair_body(p, carrys):
                sa, sb = carrys
                ga = p * (2 * G * L)
                xa = [buf[par, r, pl.ds(ga + i * L, L)] for i in range(G)]
                sa = _merge_topk(sa, _net(xa, _SORT8))
                gb = ga + G * L
                xb = [buf[par, r, pl.ds(gb + i * L, L)] for i in range(G)]
                sb = _merge_topk(sb, _net(xb, _SORT8))
                return tuple(sa), tuple(sb)

            init = (neg_v,) * K
            sa, sb = lax.fori_loop(0, NG // 2, pair_body, (init, init))

            s = _merge_topk(list(sa), list(sb))
            t01 = _msort(_sortd(s[0]), _sortd(s[1]), lane)
            t23 = _msort(_sortd(s[2]), _sortd(s[3]), lane)
            t45 = _msort(_sortd(s[4]), _sortd(s[5]), lane)
            t67 = _msort(_sortd(s[6]), _sortd(s[7]), lane)
            top8 = _msort(_msort(t01, t23, lane), _msort(t45, t67, lane),
                          lane)

            i_row = c * CR + r
            plsc.store_scatter(stage, [lane * RPW + i_row], top8, mask=lt8)
            return 0

        lax.fori_loop(0, CR, row_body, 0)

        c2 = c + 2

        @pl.when((c2 < NCH) & (par == 0))
        def _():
            copy(c2, 0, sem0).start()

        @pl.when((c2 < NCH) & (par == 1))
        def _():
            copy(c2, 1, sem1).start()

        return carry

    lax.fori_loop(0, NCH, chunk_body, 0)

    for j in range(K):
        pltpu.sync_copy(
            stage.at[pl.ds(j * RPW, RPW)],
            out_hbm.at[b_idx, j, pl.ds(d0, RPW)],
        )


@functools.partial(
    pl.kernel,
    out_type=jax.ShapeDtypeStruct((B, K, D), jnp.float32),
    mesh=plsc.VectorSubcoreMesh(core_axis_name="c", subcore_axis_name="s"),
    compiler_params=pltpu.CompilerParams(needs_layout_passes=False),
    scratch_types=[
        pltpu.VMEM((2, CR, N), jnp.float32),
        pltpu.VMEM((K * RPW,), jnp.float32),
        pltpu.SemaphoreType.DMA,
        pltpu.SemaphoreType.DMA,
    ],
)
def _sc_topk(x_hbm, out_hbm, buf, stage, sem0, sem1):
    _sc_body(x_hbm, out_hbm, buf, stage, sem0, sem1)


def kernel(inputs):
    return _sc_topk(inputs)


# speedup vs baseline: 7.4166x; 1.0055x over previous
import functools

import jax
import jax.numpy as jnp
from jax import lax
from jax.experimental import pallas as pl
from jax.experimental.pallas import tpu as pltpu
from jax.experimental.pallas import tpu_sc as plsc

K = 8
B, D, N = 4, 2048, 8192
R = B * D
L = 16
VPR = N // L
G = 8
NG = VPR // G
NC, NS = 2, 16
NW = NC * NS
RPW = R // NW
CR = 4
CW = CR * N
NCH = RPW // CR
NEG = float("-inf")

_SORT8 = [(0, 1), (2, 3), (4, 5), (6, 7),
          (0, 2), (1, 3), (4, 6), (5, 7),
          (1, 2), (5, 6), (0, 4), (3, 7),
          (1, 5), (2, 6),
          (1, 4), (3, 6),
          (2, 4), (3, 5),
          (3, 4)]
_CLEAN8 = [(0, 4), (1, 5), (2, 6), (3, 7),
           (0, 2), (1, 3), (4, 6), (5, 7),
           (0, 1), (2, 3), (4, 5), (6, 7)]


def _sortd(v):
    sk, _ = plsc.sort_key_val(v, v, descending=True)
    return sk


def _msort(a, b, lane):
    comb = jnp.where(lane < K, a, lax.rev(b, (0,)))
    return _sortd(comb)


def _net(vs, pairs):
    vs = list(vs)
    for i, j in pairs:
        hi = jnp.maximum(vs[i], vs[j])
        lo = jnp.minimum(vs[i], vs[j])
        vs[i], vs[j] = hi, lo
    return vs


def _merge_topk(s, xs):
    r = [jnp.maximum(s[i], xs[K - 1 - i]) for i in range(K)]
    return _net(r, _CLEAN8)


def _sc_body(x_hbm, out_hbm, buf, stage, sem0, sem1):
    cid = lax.axis_index("c")
    sid = lax.axis_index("s")
    w = sid * NC + cid
    row0 = w * RPW
    base_off = row0 * N
    b_idx = w // (D // RPW)
    d0 = (w % (D // RPW)) * RPW

    lane = lax.iota(jnp.int32, L)
    lt8 = lane < K
    neg_v = jnp.full((L,), NEG, jnp.float32)

    def copy(c, par, sem):
        return pltpu.make_async_copy(
            x_hbm.at[b_idx, pl.ds(d0 + c * CR, CR), :],
            buf.at[par],
            sem,
        )

    copy(0, 0, sem0).start()
    copy(1, 1, sem1).start()

    def chunk_body(c, carry):
        par = c & 1

        @pl.when(par == 0)
        def _():
            copy(c, 0, sem0).wait()

        @pl.when(par == 1)
        def _():
            copy(c, 1, sem1).wait()

        def row_body(r, _):
            def pair_body(p, carrys):
                sa, sb = carrys
                ga = p * (2 * G * L)
                xa = [buf[par, r, pl.ds(ga + i * L, L)] for i in range(G)]
                sa = _merge_topk(sa, _net(xa, _SORT8))
                gb = ga + G * L
                xb = [buf[par, r, pl.ds(gb + i * L, L)] for i in range(G)]
                sb = _merge_topk(sb, _net(xb, _SORT8))
                return tuple(sa), tuple(sb)

            init = (neg_v,) * K
            sa, sb = lax.fori_loop(0, NG // 2, pair_body, (init, init),
                                   unroll=2)

            s = _merge_topk(list(sa), list(sb))
            t01 = _msort(_sortd(s[0]), _sortd(s[1]), lane)
            t23 = _msort(_sortd(s[2]), _sortd(s[3]), lane)
            t45 = _msort(_sortd(s[4]), _sortd(s[5]), lane)
            t67 = _msort(_sortd(s[6]), _sortd(s[7]), lane)
            top8 = _msort(_msort(t01, t23, lane), _msort(t45, t67, lane),
                          lane)

            i_row = c * CR + r
            plsc.store_scatter(stage, [lane * RPW + i_row], top8, mask=lt8)
            return 0

        lax.fori_loop(0, CR, row_body, 0)

        c2 = c + 2

        @pl.when((c2 < NCH) & (par == 0))
        def _():
            copy(c2, 0, sem0).start()

        @pl.when((c2 < NCH) & (par == 1))
        def _():
            copy(c2, 1, sem1).start()

        return carry

    lax.fori_loop(0, NCH, chunk_body, 0)

    for j in range(K):
        pltpu.sync_copy(
            stage.at[pl.ds(j * RPW, RPW)],
            out_hbm.at[b_idx, j, pl.ds(d0, RPW)],
        )


@functools.partial(
    pl.kernel,
    out_type=jax.ShapeDtypeStruct((B, K, D), jnp.float32),
    mesh=plsc.VectorSubcoreMesh(core_axis_name="c", subcore_axis_name="s"),
    compiler_params=pltpu.CompilerParams(needs_layout_passes=False),
    scratch_types=[
        pltpu.VMEM((2, CR, N), jnp.float32),
        pltpu.VMEM((K * RPW,), jnp.float32),
        pltpu.SemaphoreType.DMA,
        pltpu.SemaphoreType.DMA,
    ],
)
def _sc_topk(x_hbm, out_hbm, buf, stage, sem0, sem1):
    _sc_body(x_hbm, out_hbm, buf, stage, sem0, sem1)


def kernel(inputs):
    return _sc_topk(inputs)
